# baseline (device time: 14101 ns/iter reference)
import jax
import jax.numpy as jnp
from jax import lax
from jax.experimental import pallas as pl
from jax.experimental.pallas import tpu as pltpu

N_DEV = 4
B, SQ, DM = 2, 128, 512
HQ, DH = 4, 64
GQ = 2
BLK = 64
PK = DH + 1


def kernel(x, Wq, K_ext, V_ext, Wo):

    def body(x_ref, wq_ref, k_ref, v_ref, wo_ref, out_ref,
             cl_buf, send_sems, recv_sems):
        my = lax.axis_index("i")

        barrier_sem = pltpu.get_barrier_semaphore()

        @pl.when(my != 0)
        def _():
            pl.semaphore_signal(barrier_sem, inc=1, device_id=(0,),
                                device_id_type=pl.DeviceIdType.MESH)

        @pl.when(my != 2)
        def _():
            pl.semaphore_signal(barrier_sem, inc=1, device_id=(2,),
                                device_id_type=pl.DeviceIdType.MESH)

        def partial_attn(slot, dsts):
            wqb = wq_ref[...].astype(jnp.bfloat16)
            q = [
                jnp.dot(x_ref[b].astype(jnp.bfloat16), wqb,
                        preferred_element_type=jnp.float32) * 0.125
                for b in range(B)
            ]
            for g in range(GQ):
                for b in range(B):
                    for h in range(HQ):
                        qh = q[b][g * BLK:(g + 1) * BLK,
                                  h * DH:(h + 1) * DH].astype(jnp.bfloat16)
                        kh = k_ref[b, pl.ds(g * BLK, BLK), h, :].astype(
                            jnp.bfloat16)
                        vh = v_ref[b, pl.ds(g * BLK, BLK), h, :].astype(
                            jnp.bfloat16)
                        s = lax.dot_general(
                            qh, kh, (((1,), (1,)), ((), ())),
                            preferred_element_type=jnp.float32)
                        e = jnp.exp(s)
                        l = jnp.sum(e, axis=1, keepdims=True)
                        c = jnp.dot(e.astype(jnp.bfloat16), vh,
                                    preferred_element_type=jnp.float32)
                        cl_buf[slot, b, g, h] = jnp.concatenate(
                            [c, l], axis=1).astype(jnp.bfloat16)
                if g == 0:
                    pl.semaphore_wait(barrier_sem, 3)
                for j, dst in enumerate(dsts):
                    pltpu.make_async_remote_copy(
                        src_ref=cl_buf.at[slot, :, g],
                        dst_ref=cl_buf.at[slot, :, g],
                        send_sem=send_sems.at[g * 3 + j],
                        recv_sem=recv_sems.at[slot, g],
                        device_id=(dst,),
                        device_id_type=pl.DeviceIdType.MESH,
                    ).start()

        @pl.when(my == 0)
        def _():
            partial_attn(0, (1, 2, 3))

        @pl.when(my == 2)
        def _():
            partial_attn(1, (0, 1, 3))

        def wait_recv(slot, g):
            pltpu.make_async_remote_copy(
                src_ref=cl_buf.at[slot, :, g],
                dst_ref=cl_buf.at[slot, :, g],
                send_sem=send_sems.at[0],
                recv_sem=recv_sems.at[slot, g],
                device_id=(0,),
                device_id_type=pl.DeviceIdType.MESH,
            ).wait_recv()

        wob = wo_ref[...].astype(jnp.bfloat16)
        for g in range(GQ):
            @pl.when(my != 0)
            def _():
                wait_recv(0, g)

            @pl.when(my != 2)
            def _():
                wait_recv(1, g)

            for b in range(B):
                p = (cl_buf[0, b, g].astype(jnp.float32)
                     + cl_buf[1, b, g].astype(jnp.float32))
                w3 = p[:, :, :DH] / p[:, :, DH:PK]
                ctx = jnp.concatenate(
                    [w3[h] for h in range(HQ)], axis=1).astype(jnp.bfloat16)
                out_ref[b, pl.ds(g * BLK, BLK), :] = jnp.dot(
                    ctx, wob, preferred_element_type=jnp.float32)

        @pl.when((my == 0) | (my == 2))
        def _():
            for j in range(6):
                pltpu.make_async_remote_copy(
                    src_ref=cl_buf.at[0, :, 0],
                    dst_ref=cl_buf.at[0, :, 0],
                    send_sem=send_sems.at[j],
                    recv_sem=recv_sems.at[0, 0],
                    device_id=(0,),
                    device_id_type=pl.DeviceIdType.MESH,
                ).wait_send()

    return pl.pallas_call(
        body,
        out_shape=jax.ShapeDtypeStruct((B, SQ, DM), jnp.float32),
        in_specs=[pl.BlockSpec(memory_space=pltpu.VMEM)] * 5,
        out_specs=pl.BlockSpec(memory_space=pltpu.VMEM),
        scratch_shapes=[
            pltpu.VMEM((2, B, GQ, HQ, BLK, PK), jnp.bfloat16),
            pltpu.SemaphoreType.DMA((6,)),
            pltpu.SemaphoreType.DMA((2, 2)),
        ],
        compiler_params=pltpu.CompilerParams(collective_id=0),
    )(x, Wq, K_ext, V_ext, Wo)
